# baseline (speedup 1.0000x reference)
import functools

import jax
import jax.numpy as jnp
from jax import lax
from jax.experimental import pallas as pl
from jax.experimental.pallas import tpu as pltpu
from jax.experimental.pallas import tpu_sc as plsc

B, S, D = 4, 4096, 1024
L = 16
NC, NS = 2, 16
NW = NC * NS
S_PER_W = S // NW
CH = 8
NCHUNK = S_PER_W // CH
NSLOT = 3
VECS = CH * D // L
VPR = D // L

_mesh = plsc.VectorSubcoreMesh(core_axis_name="c", subcore_axis_name="s")


@functools.partial(
    pl.kernel,
    out_type=jax.ShapeDtypeStruct((B, S, D), jnp.float32),
    mesh=_mesh,
    scratch_types=[
        pltpu.VMEM((NSLOT, CH, D), jnp.float32),
        pltpu.VMEM((NSLOT, B, CH, D), jnp.float32),
        [pltpu.SemaphoreType.DMA] * NSLOT,
        [pltpu.SemaphoreType.DMA] * NSLOT,
    ],
)
def _sc_pos_add(x_hbm, pos_hbm, out_hbm, pos_v, x_v, sem_in, sem_out):
    wid = lax.axis_index("s") * NC + lax.axis_index("c")
    s_base = wid * S_PER_W

    def start_in(ci):
        slot = ci % NSLOT
        s0 = s_base + ci * CH
        ds = [pltpu.async_copy(pos_hbm.at[pl.ds(s0, CH)],
                               pos_v.at[slot], sem_in[slot])]
        for b in range(B):
            ds.append(pltpu.async_copy(x_hbm.at[b, pl.ds(s0, CH)],
                                       x_v.at[slot, b], sem_in[slot]))
        return ds

    def start_out(ci):
        slot = ci % NSLOT
        s0 = s_base + ci * CH
        return [pltpu.async_copy(x_v.at[slot, b],
                                 out_hbm.at[b, pl.ds(s0, CH)], sem_out[slot])
                for b in range(B)]

    def compute(ci):
        slot = ci % NSLOT

        def row_body(r, _):
            @plsc.parallel_loop(0, D, L, unroll=16)
            def _vec(c):
                pv = pos_v[slot, r, pl.ds(c, L)]
                for b in range(B):
                    plsc.addupdate(x_v.at[slot, b, r, pl.ds(c, L)], pv)
            return 0

        lax.fori_loop(0, CH, row_body, 0)

    descs_in = [None] * NCHUNK
    descs_out = [None] * NCHUNK
    descs_in[0] = start_in(0)
    for ci in range(NCHUNK):
        if ci + 1 < NCHUNK:
            if ci - 2 >= 0:
                for d in descs_out[ci - 2]:
                    d.wait()
            descs_in[ci + 1] = start_in(ci + 1)
        for d in descs_in[ci]:
            d.wait()
        compute(ci)
        descs_out[ci] = start_out(ci)
    for ci in range(NCHUNK - NSLOT, NCHUNK):
        for d in descs_out[ci]:
            d.wait()


def kernel(x, pos_table):
    return _sc_pos_add(x, pos_table)


# speedup vs baseline: 1.0550x; 1.0550x over previous
import functools

import jax
import jax.numpy as jnp
from jax import lax
from jax.experimental import pallas as pl
from jax.experimental.pallas import tpu as pltpu
from jax.experimental.pallas import tpu_sc as plsc

B, S, D = 4, 4096, 1024
L = 16
NC, NS = 2, 16
NW = NC * NS
S_PER_W = S // NW
CH = 8
NCHUNK = S_PER_W // CH
NSLOT = 3
VECS = CH * D // L
VPR = D // L

_mesh = plsc.VectorSubcoreMesh(core_axis_name="c", subcore_axis_name="s")


@functools.partial(
    pl.kernel,
    out_type=jax.ShapeDtypeStruct((B, S, D), jnp.float32),
    mesh=_mesh,
    scratch_types=[
        pltpu.VMEM((NSLOT, CH, D), jnp.float32),
        pltpu.VMEM((NSLOT, B, CH, D), jnp.float32),
        [pltpu.SemaphoreType.DMA] * NSLOT,
        [pltpu.SemaphoreType.DMA] * NSLOT,
    ],
)
def _sc_pos_add(x_hbm, pos_hbm, out_hbm, pos_v, x_v, sem_in, sem_out):
    wid = lax.axis_index("s") * NC + lax.axis_index("c")
    s_base = wid * S_PER_W

    def start_in(ci):
        slot = ci % NSLOT
        s0 = s_base + ci * CH
        return [pltpu.async_copy(pos_hbm.at[pl.ds(s0, CH)],
                                 pos_v.at[slot], sem_in[slot]),
                pltpu.async_copy(x_hbm.at[:, pl.ds(s0, CH)],
                                 x_v.at[slot], sem_in[slot])]

    def start_out(ci):
        slot = ci % NSLOT
        s0 = s_base + ci * CH
        return [pltpu.async_copy(x_v.at[slot],
                                 out_hbm.at[:, pl.ds(s0, CH)], sem_out[slot])]

    def compute(ci):
        slot = ci % NSLOT

        def row_body(r, _):
            @plsc.parallel_loop(0, D, L, unroll=8)
            def _vec(c):
                pv = pos_v[slot, r, pl.ds(c, L)]
                for b in range(B):
                    plsc.addupdate(x_v.at[slot, b, r, pl.ds(c, L)], pv)
            return 0

        lax.fori_loop(0, CH, row_body, 0)

    descs_in = [None] * NCHUNK
    descs_out = [None] * NCHUNK
    descs_in[0] = start_in(0)
    for ci in range(NCHUNK):
        if ci + 1 < NCHUNK:
            if ci - 2 >= 0:
                for d in descs_out[ci - 2]:
                    d.wait()
            descs_in[ci + 1] = start_in(ci + 1)
        for d in descs_in[ci]:
            d.wait()
        compute(ci)
        descs_out[ci] = start_out(ci)
    for ci in range(NCHUNK - NSLOT, NCHUNK):
        for d in descs_out[ci]:
            d.wait()


def kernel(x, pos_table):
    return _sc_pos_add(x, pos_table)
